# baseline (speedup 1.0000x reference)
import functools

import jax
import jax.numpy as jnp
import numpy as np
from jax import lax
from jax.experimental import pallas as pl
from jax.experimental.pallas import tpu as pltpu
from jax.experimental.pallas import tpu_sc as plsc

N_NODES = 10000
D = 128

NC = 2
NS = 16
NW = NC * NS

CH = 128
K = 80
P_EDGES = NW * K * CH
NPAD = 10240
STRIPE = NPAD // NS

CHS = 128
KS = 80
IB = 4
NB = KS // IB
P_EDGES_S = NW * KS * CHS
NACC = N_NODES
SACC = NACC // NS


def _hist_body(dst_hbm, out_hbm, dst_v, ones_v, z_v, cnt_sh):
    c = lax.axis_index("c")
    s = lax.axis_index("s")
    wid = c * NS + s

    for j in range(CH // 16):
        ones_v[pl.ds(j * 16, 16)] = jnp.ones((16,), jnp.int32)

    def zb(i, carry):
        z_v[pl.ds(i * 16, 16)] = jnp.zeros((16,), jnp.int32)
        return carry

    lax.fori_loop(0, STRIPE // 16, zb, 0)
    pltpu.sync_copy(z_v, cnt_sh.at[pl.ds(s * STRIPE, STRIPE)])
    pltpu.sync_copy(dst_hbm.at[wid], dst_v)
    plsc.subcore_barrier()

    def body(k, carry):
        pltpu.sync_copy(ones_v, cnt_sh.at[dst_v.at[k]], add=True)
        return carry

    lax.fori_loop(0, K, body, 0)
    plsc.subcore_barrier()
    pltpu.sync_copy(cnt_sh.at[pl.ds(s * STRIPE, STRIPE)],
                    out_hbm.at[c, pl.ds(s * STRIPE, STRIPE)])


_HI_MASK = -65536


def _decode(rbB, rbF):
    shift = jnp.full((16,), 16, jnp.int32)
    mask = jnp.full((16,), _HI_MASK, jnp.int32)

    def row(r, carry):
        for j in range(D // 32):
            w = rbB[r, pl.ds(16 * j, 16)]
            rbF[r, pl.ds(32 * j, 16)] = lax.bitcast_convert_type(
                lax.shift_left(w, shift), jnp.float32)
            rbF[r, pl.ds(32 * j + 16, 16)] = lax.bitcast_convert_type(
                lax.bitwise_and(w, mask), jnp.float32)
        return carry

    lax.fori_loop(0, CHS, row, 0, unroll=4)


def _scatter_body(hhb_hbm, src_hbm, dst_hbm, out_hbm,
                  src_v, dst_a, dst_b, rbB0, rbB1, rbF0, rbF1,
                  acc_sh, sg0, sg1, ss0, ss1):
    c = lax.axis_index("c")
    s = lax.axis_index("s")
    wid = c * NS + s

    def zb(i, carry):
        for j in range(D // 16):
            rbF0[i, pl.ds(j * 16, 16)] = jnp.zeros((16,), jnp.float32)
            rbF1[i, pl.ds(j * 16, 16)] = jnp.zeros((16,), jnp.float32)
        return carry

    lax.fori_loop(0, CHS, zb, 0)

    def zc(t, carry):
        pltpu.sync_copy(rbF0, acc_sh.at[pl.ds(s * SACC + t * CHS, CHS)])
        return carry

    lax.fori_loop(0, SACC // CHS, zc, 0)
    pltpu.sync_copy(rbF0.at[pl.ds(0, SACC % CHS)],
                    acc_sh.at[pl.ds(s * SACC + (SACC // CHS) * CHS,
                                    SACC % CHS)])

    pltpu.sync_copy(src_hbm.at[wid, 0], src_v)
    pltpu.sync_copy(dst_hbm.at[wid, 0], dst_a)
    plsc.subcore_barrier()

    pltpu.async_copy(rbF0, acc_sh.at[dst_a.at[0]], ss0, add=True)
    pltpu.async_copy(rbF1, acc_sh.at[dst_a.at[1]], ss1, add=True)
    pltpu.async_copy(hhb_hbm.at[src_v.at[0]], rbB0, sg0)
    pltpu.async_copy(hhb_hbm.at[src_v.at[1]], rbB1, sg1)

    def process_block(dst_c, blk):
        pltpu.sync_copy(dst_hbm.at[wid, blk], dst_c)

        pltpu.make_async_copy(hhb_hbm.at[src_v.at[0]], rbB0, sg0).wait()
        pltpu.make_async_copy(rbF0, acc_sh.at[dst_c.at[0]], ss0).wait()
        _decode(rbB0, rbF0)
        pltpu.async_copy(hhb_hbm.at[src_v.at[2]], rbB0, sg0)
        pltpu.async_copy(rbF0, acc_sh.at[dst_c.at[0]], ss0, add=True)
        pltpu.make_async_copy(hhb_hbm.at[src_v.at[1]], rbB1, sg1).wait()
        pltpu.make_async_copy(rbF1, acc_sh.at[dst_c.at[1]], ss1).wait()
        _decode(rbB1, rbF1)
        pltpu.async_copy(hhb_hbm.at[src_v.at[3]], rbB1, sg1)
        pltpu.async_copy(rbF1, acc_sh.at[dst_c.at[1]], ss1, add=True)

        pltpu.make_async_copy(hhb_hbm.at[src_v.at[2]], rbB0, sg0).wait()
        pltpu.make_async_copy(rbF0, acc_sh.at[dst_c.at[2]], ss0).wait()
        _decode(rbB0, rbF0)
        pltpu.async_copy(rbF0, acc_sh.at[dst_c.at[2]], ss0, add=True)
        pltpu.make_async_copy(hhb_hbm.at[src_v.at[3]], rbB1, sg1).wait()
        pltpu.make_async_copy(rbF1, acc_sh.at[dst_c.at[3]], ss1).wait()
        _decode(rbB1, rbF1)
        pltpu.async_copy(rbF1, acc_sh.at[dst_c.at[3]], ss1, add=True)

        @pl.when(blk + 1 < NB)
        def _():
            pltpu.sync_copy(src_hbm.at[wid, blk + 1], src_v)
            pltpu.async_copy(hhb_hbm.at[src_v.at[0]], rbB0, sg0)
            pltpu.async_copy(hhb_hbm.at[src_v.at[1]], rbB1, sg1)

    def super_body(j, carry):
        process_block(dst_a, 2 * j)
        process_block(dst_b, 2 * j + 1)
        return carry

    lax.fori_loop(0, NB // 2, super_body, 0)
    pltpu.make_async_copy(rbF0, acc_sh.at[dst_b.at[0]], ss0).wait()
    pltpu.make_async_copy(rbF1, acc_sh.at[dst_b.at[1]], ss1).wait()
    plsc.subcore_barrier()
    pltpu.sync_copy(acc_sh.at[pl.ds(s * SACC, SACC)],
                    out_hbm.at[c, pl.ds(s * SACC, SACC)])


def _mm_body(x_ref, w_ref, c0_ref, c1_ref, o_ref):
    deg = (1 + c0_ref[...] + c1_ref[...]).astype(jnp.float32)
    dis = lax.rsqrt(deg)
    o_ref[...] = jnp.dot(x_ref[...], w_ref[...],
                         preferred_element_type=jnp.float32) * dis


def _epi_body(acc0_ref, acc1_ref, hh_ref, c0_ref, c1_ref, b_ref, o_ref):
    deg = (1 + c0_ref[...] + c1_ref[...]).astype(jnp.float32)
    dis = lax.rsqrt(deg)
    total = (acc0_ref[0] + acc1_ref[0] + hh_ref[...]) * dis + b_ref[...]
    o_ref[...] = jnp.maximum(total, 0.0)


def kernel(x, edge_index, W, b):
    src = edge_index[0].astype(jnp.int32)
    dst = edge_index[1].astype(jnp.int32)
    n_edges = src.shape[0]
    pad = P_EDGES - n_edges
    dst_p = jnp.concatenate(
        [dst, N_NODES + (jnp.arange(pad, dtype=jnp.int32) % (NPAD - N_NODES))])
    dst3 = dst_p.reshape(NW, K, CH)
    pad_s = P_EDGES_S - n_edges
    src_ps = jnp.concatenate(
        [src, jnp.full((pad_s,), N_NODES, jnp.int32)])
    dst_ps = jnp.concatenate(
        [dst, jnp.arange(pad_s, dtype=jnp.int32) % N_NODES])
    src4 = src_ps.reshape(NW, NB, IB, CHS)
    dst4 = dst_ps.reshape(NW, NB, IB, CHS)
    x_p = jnp.pad(x, ((0, NPAD - N_NODES), (0, 0)))

    mesh = plsc.VectorSubcoreMesh(core_axis_name="c", subcore_axis_name="s")

    hist = pl.kernel(
        _hist_body,
        out_type=jax.ShapeDtypeStruct((NC, NPAD), jnp.int32),
        mesh=mesh,
        scratch_types=[
            pltpu.VMEM((K, CH), jnp.int32),
            pltpu.VMEM((CH,), jnp.int32),
            pltpu.VMEM((STRIPE,), jnp.int32),
            pltpu.VMEM_SHARED((NPAD,), jnp.int32),
        ],
    )
    cnt = hist(dst3)
    c0 = cnt[0].reshape(NPAD, 1)
    c1 = cnt[1].reshape(NPAD, 1)

    BM = 1024
    grid = NPAD // BM
    hhat = pl.pallas_call(
        _mm_body,
        grid=(grid,),
        in_specs=[
            pl.BlockSpec((BM, D), lambda i: (i, 0)),
            pl.BlockSpec((D, D), lambda i: (0, 0)),
            pl.BlockSpec((BM, 1), lambda i: (i, 0)),
            pl.BlockSpec((BM, 1), lambda i: (i, 0)),
        ],
        out_specs=pl.BlockSpec((BM, D), lambda i: (i, 0)),
        out_shape=jax.ShapeDtypeStruct((NPAD, D), jnp.float32),
    )(x_p, W, c0, c1)

    u = jax.lax.bitcast_convert_type(hhat.astype(jnp.bfloat16), jnp.uint16)
    j = np.arange(D // 2)
    perm_lo = 32 * (j // 16) + (j % 16)
    lo = u[:, perm_lo].astype(jnp.uint32)
    hi = u[:, perm_lo + 16].astype(jnp.uint32)
    hhb = jax.lax.bitcast_convert_type(lo | (hi << 16), jnp.int32)

    scatter = pl.kernel(
        _scatter_body,
        out_type=jax.ShapeDtypeStruct((NC, NACC, D), jnp.float32),
        mesh=mesh,
        compiler_params=pltpu.CompilerParams(use_tc_tiling_on_sc=False),
        scratch_types=[
            pltpu.VMEM((IB, CHS), jnp.int32),
            pltpu.VMEM((IB, CHS), jnp.int32),
            pltpu.VMEM((IB, CHS), jnp.int32),
            pltpu.VMEM((CHS, D // 2), jnp.int32),
            pltpu.VMEM((CHS, D // 2), jnp.int32),
            pltpu.VMEM((CHS, D), jnp.float32),
            pltpu.VMEM((CHS, D), jnp.float32),
            pltpu.VMEM_SHARED((NACC, D), jnp.float32),
            pltpu.SemaphoreType.DMA,
            pltpu.SemaphoreType.DMA,
            pltpu.SemaphoreType.DMA,
            pltpu.SemaphoreType.DMA,
        ],
    )
    acc = scatter(hhb, src4, dst4)

    b2 = b.reshape(1, D)
    BE = 1000
    out = pl.pallas_call(
        _epi_body,
        grid=(N_NODES // BE,),
        in_specs=[
            pl.BlockSpec((1, BE, D), lambda i: (0, i, 0)),
            pl.BlockSpec((1, BE, D), lambda i: (1, i, 0)),
            pl.BlockSpec((BE, D), lambda i: (i, 0)),
            pl.BlockSpec((BE, 1), lambda i: (i, 0)),
            pl.BlockSpec((BE, 1), lambda i: (i, 0)),
            pl.BlockSpec((1, D), lambda i: (0, 0)),
        ],
        out_specs=pl.BlockSpec((BE, D), lambda i: (i, 0)),
        out_shape=jax.ShapeDtypeStruct((N_NODES, D), jnp.float32),
    )(acc, acc, hhat, c0, c1, b2)

    return out


# speedup vs baseline: 1.1248x; 1.1248x over previous
import functools

import jax
import jax.numpy as jnp
import numpy as np
from jax import lax
from jax.experimental import pallas as pl
from jax.experimental.pallas import tpu as pltpu
from jax.experimental.pallas import tpu_sc as plsc

N_NODES = 10000
D = 128

NC = 2
NS = 16
NW = NC * NS

CH = 128
K = 80
P_EDGES = NW * K * CH
NPAD = 10240
STRIPE = NPAD // NS

CHS = 128
KA = 88
KB = 72
IB = 8
NBA = KA // IB
NBB = KB // IB
P_EDGES_S = NS * (KA + KB) * CHS
NACC = N_NODES
SACC = NACC // NS


def _hist_body(dst_hbm, out_hbm, dst_v, ones_v, z_v, cnt_sh):
    c = lax.axis_index("c")
    s = lax.axis_index("s")
    wid = c * NS + s

    for j in range(CH // 16):
        ones_v[pl.ds(j * 16, 16)] = jnp.ones((16,), jnp.int32)

    def zb(i, carry):
        z_v[pl.ds(i * 16, 16)] = jnp.zeros((16,), jnp.int32)
        return carry

    lax.fori_loop(0, STRIPE // 16, zb, 0)
    pltpu.sync_copy(z_v, cnt_sh.at[pl.ds(s * STRIPE, STRIPE)])
    pltpu.sync_copy(dst_hbm.at[wid], dst_v)
    plsc.subcore_barrier()

    def body(k, carry):
        pltpu.sync_copy(ones_v, cnt_sh.at[dst_v.at[k]], add=True)
        return carry

    lax.fori_loop(0, K, body, 0)
    plsc.subcore_barrier()
    pltpu.sync_copy(cnt_sh.at[pl.ds(s * STRIPE, STRIPE)],
                    out_hbm.at[c, pl.ds(s * STRIPE, STRIPE)])


_HI_MASK = -65536


def _decode(rbB, rbF):
    shift = jnp.full((16,), 16, jnp.int32)
    mask = jnp.full((16,), _HI_MASK, jnp.int32)

    def row(r, carry):
        for j in range(D // 32):
            w = rbB[r, pl.ds(16 * j, 16)]
            rbF[r, pl.ds(32 * j, 16)] = lax.bitcast_convert_type(
                lax.shift_left(w, shift), jnp.float32)
            rbF[r, pl.ds(32 * j + 16, 16)] = lax.bitcast_convert_type(
                lax.bitwise_and(w, mask), jnp.float32)
        return carry

    lax.fori_loop(0, CHS, row, 0)


def _edge_loop(hhb_hbm, src_hbm, dst_hbm, s, nb,
               src_v, dst_v, rbB0, rbB1, rbF, acc_sh, sg0, sg1):
    def blk_body(blk, carry):
        pltpu.sync_copy(src_hbm.at[s, blk], src_v)
        pltpu.sync_copy(dst_hbm.at[s, blk], dst_v)
        pltpu.async_copy(hhb_hbm.at[src_v.at[0]], rbB0, sg0)
        pltpu.async_copy(hhb_hbm.at[src_v.at[1]], rbB1, sg1)

        def body(j, carry2):
            k0 = 2 * j
            pltpu.make_async_copy(hhb_hbm.at[src_v.at[0]], rbB0, sg0).wait()
            _decode(rbB0, rbF)
            pltpu.async_copy(hhb_hbm.at[src_v.at[k0 + 2]], rbB0, sg0)
            pltpu.sync_copy(rbF, acc_sh.at[dst_v.at[k0]], add=True)
            pltpu.make_async_copy(hhb_hbm.at[src_v.at[1]], rbB1, sg1).wait()
            _decode(rbB1, rbF)
            pltpu.async_copy(hhb_hbm.at[src_v.at[k0 + 3]], rbB1, sg1)
            pltpu.sync_copy(rbF, acc_sh.at[dst_v.at[k0 + 1]], add=True)
            return carry2

        lax.fori_loop(0, IB // 2 - 1, body, 0)
        pltpu.make_async_copy(hhb_hbm.at[src_v.at[0]], rbB0, sg0).wait()
        _decode(rbB0, rbF)
        pltpu.sync_copy(rbF, acc_sh.at[dst_v.at[IB - 2]], add=True)
        pltpu.make_async_copy(hhb_hbm.at[src_v.at[1]], rbB1, sg1).wait()
        _decode(rbB1, rbF)
        pltpu.sync_copy(rbF, acc_sh.at[dst_v.at[IB - 1]], add=True)
        return carry

    lax.fori_loop(0, nb, blk_body, 0)


def _scatter_body(hhb_hbm, srcA_hbm, dstA_hbm, srcB_hbm, dstB_hbm, out_hbm,
                  src_v, dst_v, rbB0, rbB1, rbF, acc_sh, sg0, sg1):
    c = lax.axis_index("c")
    s = lax.axis_index("s")

    def zb(i, carry):
        for j in range(D // 16):
            rbF[i, pl.ds(j * 16, 16)] = jnp.zeros((16,), jnp.float32)
        return carry

    lax.fori_loop(0, CHS, zb, 0)

    def zc(t, carry):
        pltpu.sync_copy(rbF, acc_sh.at[pl.ds(s * SACC + t * CHS, CHS)])
        return carry

    lax.fori_loop(0, SACC // CHS, zc, 0)
    pltpu.sync_copy(rbF.at[pl.ds(0, SACC % CHS)],
                    acc_sh.at[pl.ds(s * SACC + (SACC // CHS) * CHS,
                                    SACC % CHS)])
    plsc.subcore_barrier()

    @pl.when(c == 0)
    def _():
        _edge_loop(hhb_hbm, srcA_hbm, dstA_hbm, s, NBA,
                   src_v, dst_v, rbB0, rbB1, rbF, acc_sh, sg0, sg1)

    @pl.when(c == 1)
    def _():
        _edge_loop(hhb_hbm, srcB_hbm, dstB_hbm, s, NBB,
                   src_v, dst_v, rbB0, rbB1, rbF, acc_sh, sg0, sg1)

    plsc.subcore_barrier()
    pltpu.sync_copy(acc_sh.at[pl.ds(s * SACC, SACC)],
                    out_hbm.at[c, pl.ds(s * SACC, SACC)])


def _mm_body(x_ref, w_ref, c0_ref, c1_ref, o_ref):
    deg = (1 + c0_ref[...] + c1_ref[...]).astype(jnp.float32)
    dis = lax.rsqrt(deg)
    o_ref[...] = jnp.dot(x_ref[...], w_ref[...],
                         preferred_element_type=jnp.float32) * dis


def _epi_body(acc0_ref, acc1_ref, hh_ref, c0_ref, c1_ref, b_ref, o_ref):
    deg = (1 + c0_ref[...] + c1_ref[...]).astype(jnp.float32)
    dis = lax.rsqrt(deg)
    total = (acc0_ref[0] + acc1_ref[0] + hh_ref[...]) * dis + b_ref[...]
    o_ref[...] = jnp.maximum(total, 0.0)


def kernel(x, edge_index, W, b):
    src = edge_index[0].astype(jnp.int32)
    dst = edge_index[1].astype(jnp.int32)
    n_edges = src.shape[0]
    pad = P_EDGES - n_edges
    dst_p = jnp.concatenate(
        [dst, N_NODES + (jnp.arange(pad, dtype=jnp.int32) % (NPAD - N_NODES))])
    dst3 = dst_p.reshape(NW, K, CH)
    pad_s = P_EDGES_S - n_edges
    src_ps = jnp.concatenate(
        [src, jnp.full((pad_s,), N_NODES, jnp.int32)])
    dst_ps = jnp.concatenate(
        [dst, jnp.arange(pad_s, dtype=jnp.int32) % N_NODES])
    ea = NS * KA * CHS
    srcA = src_ps[:ea].reshape(NS, NBA, IB, CHS)
    dstA = dst_ps[:ea].reshape(NS, NBA, IB, CHS)
    srcB = src_ps[ea:].reshape(NS, NBB, IB, CHS)
    dstB = dst_ps[ea:].reshape(NS, NBB, IB, CHS)
    x_p = jnp.pad(x, ((0, NPAD - N_NODES), (0, 0)))

    mesh = plsc.VectorSubcoreMesh(core_axis_name="c", subcore_axis_name="s")

    hist = pl.kernel(
        _hist_body,
        out_type=jax.ShapeDtypeStruct((NC, NPAD), jnp.int32),
        mesh=mesh,
        scratch_types=[
            pltpu.VMEM((K, CH), jnp.int32),
            pltpu.VMEM((CH,), jnp.int32),
            pltpu.VMEM((STRIPE,), jnp.int32),
            pltpu.VMEM_SHARED((NPAD,), jnp.int32),
        ],
    )
    cnt = hist(dst3)
    c0 = cnt[0].reshape(NPAD, 1)
    c1 = cnt[1].reshape(NPAD, 1)

    BM = 1024
    grid = NPAD // BM
    hhat = pl.pallas_call(
        _mm_body,
        grid=(grid,),
        in_specs=[
            pl.BlockSpec((BM, D), lambda i: (i, 0)),
            pl.BlockSpec((D, D), lambda i: (0, 0)),
            pl.BlockSpec((BM, 1), lambda i: (i, 0)),
            pl.BlockSpec((BM, 1), lambda i: (i, 0)),
        ],
        out_specs=pl.BlockSpec((BM, D), lambda i: (i, 0)),
        out_shape=jax.ShapeDtypeStruct((NPAD, D), jnp.float32),
    )(x_p, W, c0, c1)

    u = jax.lax.bitcast_convert_type(hhat.astype(jnp.bfloat16), jnp.uint16)
    j = np.arange(D // 2)
    perm_lo = 32 * (j // 16) + (j % 16)
    lo = u[:, perm_lo].astype(jnp.uint32)
    hi = u[:, perm_lo + 16].astype(jnp.uint32)
    hhb = jax.lax.bitcast_convert_type(lo | (hi << 16), jnp.int32)

    scatter = pl.kernel(
        _scatter_body,
        out_type=jax.ShapeDtypeStruct((NC, NACC, D), jnp.float32),
        mesh=mesh,
        compiler_params=pltpu.CompilerParams(use_tc_tiling_on_sc=False),
        scratch_types=[
            pltpu.VMEM((IB, CHS), jnp.int32),
            pltpu.VMEM((IB, CHS), jnp.int32),
            pltpu.VMEM((CHS, D // 2), jnp.int32),
            pltpu.VMEM((CHS, D // 2), jnp.int32),
            pltpu.VMEM((CHS, D), jnp.float32),
            pltpu.VMEM_SHARED((NACC, D), jnp.float32),
            pltpu.SemaphoreType.DMA,
            pltpu.SemaphoreType.DMA,
        ],
    )
    acc = scatter(hhb, srcA, dstA, srcB, dstB)

    b2 = b.reshape(1, D)
    BE = 1000
    out = pl.pallas_call(
        _epi_body,
        grid=(N_NODES // BE,),
        in_specs=[
            pl.BlockSpec((1, BE, D), lambda i: (0, i, 0)),
            pl.BlockSpec((1, BE, D), lambda i: (1, i, 0)),
            pl.BlockSpec((BE, D), lambda i: (i, 0)),
            pl.BlockSpec((BE, 1), lambda i: (i, 0)),
            pl.BlockSpec((BE, 1), lambda i: (i, 0)),
            pl.BlockSpec((1, D), lambda i: (0, 0)),
        ],
        out_specs=pl.BlockSpec((BE, D), lambda i: (i, 0)),
        out_shape=jax.ShapeDtypeStruct((N_NODES, D), jnp.float32),
    )(acc, acc, hhat, c0, c1, b2)

    return out
